# baseline (speedup 1.0000x reference)
import jax
import jax.numpy as jnp
from jax.experimental import pallas as pl
from jax.experimental.pallas import tpu as pltpu

NUM_USERS = 25000
NUM_ITEMS = 25000
N = NUM_USERS + NUM_ITEMS
N_LAYERS = 3
B = 4096


def _loss_body(u_ref, p_ref, n_ref, out_ref):
    u = u_ref[...]
    d = p_ref[...] - n_ref[...]
    x = jnp.sum(u * d, axis=1)
    ls = jnp.minimum(x, 0.0) - jnp.log1p(jnp.exp(-jnp.abs(x)))
    out_ref[...] = jnp.reshape(-jnp.mean(ls), (1, 1))


def _loss(u, p, n):
    out = pl.pallas_call(
        _loss_body,
        out_shape=jax.ShapeDtypeStruct((1, 1), jnp.float32),
    )(u, p, n)
    return out[0, 0]


def kernel(user_ids, pos_item_ids, neg_item_ids, e_u1, e_u2, e_i1, e_i2, adj_row, adj_col, adj_vals):
    def propagate(u_tab, i_tab):
        all_emb = jnp.concatenate([u_tab, i_tab], axis=0)
        embs = [all_emb]
        for _ in range(N_LAYERS):
            msgs = embs[-1][adj_col] * adj_vals[:, None]
            nxt = jax.ops.segment_sum(msgs, adj_row, num_segments=N)
            embs.append(nxt)
        m = jnp.mean(jnp.stack(embs, axis=0), axis=0)
        return m[:NUM_USERS], m[NUM_USERS:]

    u_emb_1, i_emb_1 = propagate(e_u1, e_i1)
    u_emb_2, i_emb_2 = propagate(e_u2, e_i2)
    user_latent = jnp.concatenate([u_emb_1[user_ids], u_emb_2[user_ids]], axis=1)
    pos_item_latent = jnp.concatenate([i_emb_1[pos_item_ids], i_emb_2[pos_item_ids]], axis=1)
    neg_item_latent = jnp.concatenate([i_emb_1[neg_item_ids], i_emb_2[neg_item_ids]], axis=1)
    return _loss(user_latent, pos_item_latent, neg_item_latent)


# speedup vs baseline: 9.8044x; 9.8044x over previous
import functools

import jax
import jax.numpy as jnp
from jax import lax
from jax.experimental import pallas as pl
from jax.experimental.pallas import tpu as pltpu
import jax.experimental.pallas.tpu_sc as plsc

NU = 25000
NN = 50000
HALF = 400000
D = 64
W = 128
NWIN = 200
PER_TILE = W * NWIN
PER_SC = 16 * PER_TILE
PAD = PER_SC - HALF
NUP = 25088
STRIPE = 1568
NRP = 16 * STRIPE
NBUF = 2
WBLK = 40
NBLK = NWIN // WBLK
B = 4096


def _mesh():
    return plsc.VectorSubcoreMesh(core_axis_name="c", subcore_axis_name="s",
                                  num_cores=2, num_subcores=16)


def _deg_body(dstp, ones_h, zvec, degout, acc1, didx, ones_v, ssem):
    c = lax.axis_index("c")
    t = lax.axis_index("s")
    pltpu.sync_copy(zvec.at[pl.ds(t * STRIPE, STRIPE)],
                    acc1.at[pl.ds(t * STRIPE, STRIPE)])
    pltpu.sync_copy(ones_h, ones_v)
    plsc.subcore_barrier()

    def blk_body(blk, carry):
        pltpu.sync_copy(dstp.at[c, t, pl.ds(blk * WBLK, WBLK)], didx)

        def outer(o, carry2):
            for b in range(NBUF):
                j = o * NBUF + b
                pltpu.async_copy(ones_v, acc1.at[didx.at[j]], ssem.at[b],
                                 add=True)
                pltpu.make_async_copy(ones_v, acc1.at[didx.at[j]],
                                      ssem.at[b]).wait()
            return carry2

        return lax.fori_loop(0, WBLK // NBUF, outer, carry)

    lax.fori_loop(0, NBLK, blk_body, 0)
    plsc.subcore_barrier()

    @pl.when(t == 0)
    def _drain():
        pltpu.sync_copy(acc1, degout.at[c])


def _deg_call(dstp, ones_h, zvec):
    return pl.kernel(
        _deg_body,
        out_type=jax.ShapeDtypeStruct((2, NUP), jnp.float32),
        mesh=_mesh(),
        compiler_params=pltpu.CompilerParams(use_tc_tiling_on_sc=False),
        scratch_types=[
            pltpu.VMEM_SHARED((NRP,), jnp.float32),
            pltpu.VMEM((WBLK, W), jnp.int32),
            pltpu.VMEM((W,), jnp.float32),
            pltpu.SemaphoreType.DMA((NBUF,)),
        ],
    )(dstp, ones_h, zvec)


def _layer_body(gprev, srcp, dstp, zrows, gout, acc, sidx, didx, rbuf, gsem, ssem):
    c = lax.axis_index("c")
    t = lax.axis_index("s")
    for chunk in range(2):
        pltpu.sync_copy(zrows.at[pl.ds(t * STRIPE, STRIPE)],
                        acc.at[pl.ds(t * STRIPE, STRIPE)])
        plsc.subcore_barrier()

        def blk_body(blk, carry):
            pltpu.sync_copy(srcp.at[chunk, c, t, pl.ds(blk * WBLK, WBLK)],
                            sidx)
            pltpu.sync_copy(dstp.at[c, t, pl.ds(blk * WBLK, WBLK)], didx)
            for b in range(NBUF):
                pltpu.async_copy(gprev.at[sidx.at[b]], rbuf.at[b], gsem.at[b])

            def outer(o, carry2):
                for b in range(NBUF):
                    j = o * NBUF + b
                    pltpu.make_async_copy(gprev.at[sidx.at[j]], rbuf.at[b],
                                          gsem.at[b]).wait()
                    pltpu.async_copy(rbuf.at[b], acc.at[didx.at[j]],
                                     ssem.at[b], add=True)
                    pltpu.make_async_copy(rbuf.at[b], acc.at[didx.at[j]],
                                          ssem.at[b]).wait()

                    @pl.when(j + NBUF < WBLK)
                    def _next():
                        pltpu.async_copy(gprev.at[sidx.at[j + NBUF]],
                                         rbuf.at[b], gsem.at[b])
                return carry2

            return lax.fori_loop(0, WBLK // NBUF, outer, carry)

        lax.fori_loop(0, NBLK, blk_body, 0)
        plsc.subcore_barrier()
        base = chunk * NN + c * NU + t * STRIPE

        @pl.when(t < 15)
        def _drain():
            pltpu.sync_copy(acc.at[pl.ds(t * STRIPE, STRIPE)],
                            gout.at[pl.ds(base, STRIPE)])

        @pl.when(t == 15)
        def _drain_last():
            pltpu.sync_copy(acc.at[pl.ds(t * STRIPE, NU - 15 * STRIPE)],
                            gout.at[pl.ds(base, NU - 15 * STRIPE)])


def _layer_call(gprev, srcp, dstp, zrows):
    return pl.kernel(
        _layer_body,
        out_type=jax.ShapeDtypeStruct((2 * NN, D), jnp.float32),
        mesh=_mesh(),
        compiler_params=pltpu.CompilerParams(use_tc_tiling_on_sc=False),
        scratch_types=[
            pltpu.VMEM_SHARED((NRP, D), jnp.float32),
            pltpu.VMEM((WBLK, W), jnp.int32),
            pltpu.VMEM((WBLK, W), jnp.int32),
            pltpu.VMEM((NBUF, W, D), jnp.float32),
            pltpu.SemaphoreType.DMA((NBUF,)),
            pltpu.SemaphoreType.DMA((NBUF,)),
        ],
    )(gprev, srcp, dstp, zrows)


def _fgather_body(stab, fidx, out, fidx_v, rbuf):
    c = lax.axis_index("c")
    t = lax.axis_index("s")
    wid = c * 16 + t
    pltpu.sync_copy(fidx.at[wid], fidx_v)
    for j in range(6):
        pltpu.sync_copy(stab.at[fidx_v.at[j]], rbuf)
        pltpu.sync_copy(rbuf, out.at[pl.ds(wid * 6 * W + j * W, W)])


def _fgather_call(stab, fidx):
    return pl.kernel(
        _fgather_body,
        out_type=jax.ShapeDtypeStruct((6 * B, D), jnp.float32),
        mesh=_mesh(),
        compiler_params=pltpu.CompilerParams(use_tc_tiling_on_sc=False),
        scratch_types=[
            pltpu.VMEM((6, W), jnp.int32),
            pltpu.VMEM((W, D), jnp.float32),
        ],
    )(stab, fidx)


def _rowscale_body(x_ref, v_ref, o_ref):
    o_ref[...] = x_ref[...] * v_ref[...]


def _rowscale(x, v):
    return pl.pallas_call(
        _rowscale_body,
        grid=(10,),
        in_specs=[pl.BlockSpec((10000, D), lambda i: (i, 0)),
                  pl.BlockSpec((10000, 1), lambda i: (i, 0))],
        out_specs=pl.BlockSpec((10000, D), lambda i: (i, 0)),
        out_shape=jax.ShapeDtypeStruct((2 * NN, D), jnp.float32),
    )(x, v)


def _sum4_body(a_ref, b_ref, c_ref, d_ref, v_ref, o_ref):
    o_ref[...] = (a_ref[...] + b_ref[...] + c_ref[...] + d_ref[...]) * v_ref[...]


def _sum4scale(a, b, c, d, v):
    return pl.pallas_call(
        _sum4_body,
        grid=(10,),
        in_specs=[pl.BlockSpec((10000, D), lambda i: (i, 0))] * 4
        + [pl.BlockSpec((10000, 1), lambda i: (i, 0))],
        out_specs=pl.BlockSpec((10000, D), lambda i: (i, 0)),
        out_shape=jax.ShapeDtypeStruct((2 * NN, D), jnp.float32),
    )(a, b, c, d, v)


def _loss_body(g_ref, o_ref):
    g = g_ref[...]
    u1, p1, n1 = g[0:B], g[B:2 * B], g[2 * B:3 * B]
    u2, p2, n2 = g[3 * B:4 * B], g[4 * B:5 * B], g[5 * B:6 * B]
    x = jnp.sum(u1 * (p1 - n1) + u2 * (p2 - n2), axis=1)
    ls = jnp.minimum(x, 0.0) - jnp.log1p(jnp.exp(-jnp.abs(x)))
    o_ref[...] = jnp.reshape(-jnp.mean(ls), (1, 1))


def _loss(g):
    out = pl.pallas_call(
        _loss_body,
        out_shape=jax.ShapeDtypeStruct((1, 1), jnp.float32),
    )(g)
    return out[0, 0]


def kernel(user_ids, pos_item_ids, neg_item_ids, e_u1, e_u2, e_i1, e_i2,
           adj_row, adj_col, adj_vals):
    i32 = jnp.int32
    f32 = jnp.float32
    E = jnp.concatenate([e_u1, e_i1, e_u2, e_i2], axis=0)

    pidx = jnp.arange(PAD, dtype=i32)
    pad_src = pidx % NN
    pad_dst = NU + pidx % (NRP - NU)
    src = jnp.concatenate([adj_col[:HALF].astype(i32), pad_src,
                           adj_col[HALF:].astype(i32), pad_src]).reshape(2, PER_SC)
    srcp = jnp.stack([src, src + NN]).reshape(2, 2, 16, NWIN, W)
    dst = jnp.concatenate([adj_row[:HALF].astype(i32), pad_dst,
                           adj_row[HALF:].astype(i32) - NU, pad_dst])
    dstp = dst.reshape(2, 16, NWIN, W)

    ones_h = jnp.ones((W,), f32)
    zvec = jnp.zeros((NRP,), f32)
    zrows = jnp.zeros((NRP, D), f32)

    deg2 = _deg_call(dstp, ones_h, zvec)
    deg = jnp.maximum(deg2[:, :NU].reshape(NN), 1.0)
    s1 = lax.rsqrt(deg)
    s2 = jnp.concatenate([s1, s1]).reshape(2 * NN, 1)
    dinv2 = jnp.concatenate([1.0 / deg, 1.0 / deg]).reshape(2 * NN, 1)
    q2 = jnp.concatenate([jnp.sqrt(deg), jnp.sqrt(deg)]).reshape(2 * NN, 1) * 0.25

    g = _rowscale(E, s2)
    tabs = [g]
    for _ in range(3):
        raw = _layer_call(tabs[-1], srcp, dstp, zrows)
        tabs.append(_rowscale(raw, dinv2))
    stab = _sum4scale(tabs[0], tabs[1], tabs[2], tabs[3], q2)

    idx_all = jnp.concatenate([
        user_ids.astype(i32), NU + pos_item_ids.astype(i32),
        NU + neg_item_ids.astype(i32), NN + user_ids.astype(i32),
        NN + NU + pos_item_ids.astype(i32), NN + NU + neg_item_ids.astype(i32)])
    fidx = idx_all.reshape(32, 6, W)
    gathered = _fgather_call(stab, fidx)
    return _loss(gathered)


# speedup vs baseline: 10.6136x; 1.0825x over previous
import jax
import jax.numpy as jnp
from jax import lax
from jax.experimental import pallas as pl
from jax.experimental.pallas import tpu as pltpu
import jax.experimental.pallas.tpu_sc as plsc

NU = 25000
NN = 50000
HALF = 400000
D = 64
W = 128
NWIN = 200
PER_TILE = W * NWIN
PER_SC = 16 * PER_TILE
PAD = PER_SC - HALF
STRIPE = 1568
NRP = 16 * STRIPE
NBUF = 2
WBLK = 20
NBLK = NWIN // WBLK
IBLK = WBLK * W
B = 4096


def _mesh():
    return plsc.VectorSubcoreMesh(core_axis_name="c", subcore_axis_name="s",
                                  num_cores=2, num_subcores=16)


def _sc_params():
    return pltpu.CompilerParams(use_tc_tiling_on_sc=False,
                                needs_layout_passes=False)


def _deg_body(dstp, degout, acc1, didx, ones_v, zbuf, ssem):
    c = lax.axis_index("c")
    t = lax.axis_index("s")
    vone = jnp.full((16,), 1.0, jnp.float32)
    vzero = jnp.zeros((16,), jnp.float32)

    def fill(i, carry):
        ones_v[pl.ds(i * 16, 16)] = vone
        zbuf[pl.ds(i * 16, 16)] = vzero
        return carry

    lax.fori_loop(0, W // 16, fill, 0)
    for z in range(12):
        pltpu.sync_copy(zbuf, acc1.at[pl.ds(t * STRIPE + z * W, W)])
    pltpu.sync_copy(zbuf.at[pl.ds(0, STRIPE - 12 * W)],
                    acc1.at[pl.ds(t * STRIPE + 12 * W, STRIPE - 12 * W)])
    plsc.subcore_barrier()
    ebase = c * PER_SC + t * PER_TILE

    def blk_body(blk, carry):
        pltpu.sync_copy(dstp.at[pl.ds(ebase + blk * IBLK, IBLK)], didx)

        def outer(o, carry2):
            for b in range(NBUF):
                j = o * NBUF + b
                idxw = didx.at[pl.ds(j * W, W)]
                pltpu.async_copy(ones_v, acc1.at[idxw], ssem.at[b], add=True)
                pltpu.make_async_copy(ones_v, acc1.at[idxw], ssem.at[b]).wait()
            return carry2

        return lax.fori_loop(0, WBLK // NBUF, outer, carry)

    lax.fori_loop(0, NBLK, blk_body, 0)
    plsc.subcore_barrier()

    @pl.when(t == 0)
    def _drain():
        pltpu.sync_copy(acc1.at[pl.ds(0, NU)], degout.at[pl.ds(c * NU, NU)])


def _deg_call(dstp):
    return pl.kernel(
        _deg_body,
        out_type=jax.ShapeDtypeStruct((NN,), jnp.float32),
        mesh=_mesh(),
        compiler_params=_sc_params(),
        scratch_types=[
            pltpu.VMEM_SHARED((NRP,), jnp.float32),
            pltpu.VMEM((IBLK,), jnp.int32),
            pltpu.VMEM((W,), jnp.float32),
            pltpu.VMEM((W,), jnp.float32),
            pltpu.SemaphoreType.DMA((NBUF,)),
        ],
    )(dstp)


def _layer_body(g0, g1, srcp, dstp, deg, out0, out1,
                acc, sidx, didx, degw, rbuf, gsem, ssem):
    c = lax.axis_index("c")
    t = lax.axis_index("s")
    ebase = c * PER_SC + t * PER_TILE
    rbase = c * NU + t * STRIPE
    vzero = jnp.zeros((16,), jnp.float32)
    for chunk in range(2):
        gprev = (g0, g1)[chunk]
        gout = (out0, out1)[chunk]

        def zfill(i, carry):
            for f in range(4):
                rbuf[1, i, pl.ds(f * 16, 16)] = vzero
            return carry

        lax.fori_loop(0, W, zfill, 0)
        for z in range(12):
            pltpu.sync_copy(rbuf.at[1], acc.at[pl.ds(t * STRIPE + z * W, W)])
        pltpu.sync_copy(rbuf.at[1, pl.ds(0, STRIPE - 12 * W)],
                        acc.at[pl.ds(t * STRIPE + 12 * W, STRIPE - 12 * W)])
        plsc.subcore_barrier()

        def blk_body(blk, carry):
            pltpu.sync_copy(srcp.at[pl.ds(ebase + blk * IBLK, IBLK)], sidx)
            pltpu.sync_copy(dstp.at[pl.ds(ebase + blk * IBLK, IBLK)], didx)
            for b in range(NBUF):
                pltpu.async_copy(gprev.at[sidx.at[pl.ds(b * W, W)]],
                                 rbuf.at[b], gsem.at[b])

            def outer(o, carry2):
                for b in range(NBUF):
                    j = o * NBUF + b
                    sw = sidx.at[pl.ds(j * W, W)]
                    dw = didx.at[pl.ds(j * W, W)]
                    pltpu.make_async_copy(gprev.at[sw], rbuf.at[b],
                                          gsem.at[b]).wait()
                    pltpu.async_copy(rbuf.at[b], acc.at[dw], ssem.at[b],
                                     add=True)
                    pltpu.make_async_copy(rbuf.at[b], acc.at[dw],
                                          ssem.at[b]).wait()

                    @pl.when(j + NBUF < WBLK)
                    def _next():
                        pltpu.async_copy(
                            gprev.at[sidx.at[pl.ds((j + NBUF) * W, W)]],
                            rbuf.at[b], gsem.at[b])
                return carry2

            return lax.fori_loop(0, WBLK // NBUF, outer, carry)

        lax.fori_loop(0, NBLK, blk_body, 0)
        plsc.subcore_barrier()

        def drain_block(nrows, row0):
            pltpu.sync_copy(acc.at[pl.ds(t * STRIPE + row0, nrows)],
                            rbuf.at[0, pl.ds(0, nrows)])
            pltpu.sync_copy(deg.at[pl.ds(rbase + row0, nrows)],
                            degw.at[pl.ds(0, nrows)])

            def recip(i, carry3):
                degw[pl.ds(i * 16, 16)] = 1.0 / degw[pl.ds(i * 16, 16)]
                return carry3

            lax.fori_loop(0, W // 16, recip, 0)

            def rows(r, carry3):
                dv = plsc.load_gather(
                    degw, [jnp.full((16,), r, jnp.int32)])
                for f in range(4):
                    rbuf[0, r, pl.ds(f * 16, 16)] = (
                        rbuf[0, r, pl.ds(f * 16, 16)] * dv)
                return carry3

            lax.fori_loop(0, nrows, rows, 0)
            pltpu.sync_copy(rbuf.at[0, pl.ds(0, nrows)],
                            gout.at[pl.ds(rbase + row0, nrows)])

        @pl.when(t < 15)
        def _drain():
            for blkr in range(13):
                nr = W if blkr < 12 else STRIPE - 12 * W
                drain_block(nr, blkr * W)

        @pl.when(t == 15)
        def _drain_last():
            for blkr in range(12):
                nr = W if blkr < 11 else 1480 - 11 * W
                drain_block(nr, blkr * W)


def _layer_call(g0, g1, srcp, dstp, deg):
    return pl.kernel(
        _layer_body,
        out_type=(jax.ShapeDtypeStruct((NN, D), jnp.float32),
                  jax.ShapeDtypeStruct((NN, D), jnp.float32)),
        mesh=_mesh(),
        compiler_params=_sc_params(),
        scratch_types=[
            pltpu.VMEM_SHARED((NRP, D), jnp.float32),
            pltpu.VMEM((IBLK,), jnp.int32),
            pltpu.VMEM((IBLK,), jnp.int32),
            pltpu.VMEM((W,), jnp.float32),
            pltpu.VMEM((NBUF, W, D), jnp.float32),
            pltpu.SemaphoreType.DMA((NBUF,)),
            pltpu.SemaphoreType.DMA((NBUF,)),
        ],
    )(g0, g1, srcp, dstp, deg)


def _fgather_body(s0, s1, fidx, out, fidx_v, rbuf):
    c = lax.axis_index("c")
    t = lax.axis_index("s")
    wid = c * 16 + t
    pltpu.sync_copy(fidx.at[wid], fidx_v)
    for chunk in range(2):
        stab = (s0, s1)[chunk]
        for role in range(3):
            pltpu.sync_copy(stab.at[fidx_v.at[role]], rbuf)
            obase = (chunk * 3 + role) * B + wid * W
            pltpu.sync_copy(rbuf, out.at[pl.ds(obase, W)])


def _fgather_call(s0, s1, fidx):
    return pl.kernel(
        _fgather_body,
        out_type=jax.ShapeDtypeStruct((6 * B, D), jnp.float32),
        mesh=_mesh(),
        compiler_params=_sc_params(),
        scratch_types=[
            pltpu.VMEM((3, W), jnp.int32),
            pltpu.VMEM((W, D), jnp.float32),
        ],
    )(s0, s1, fidx)


def _rowscale_body(x_ref, v_ref, o_ref):
    o_ref[...] = x_ref[...] * v_ref[...]


def _rowscale(x, v):
    return pl.pallas_call(
        _rowscale_body,
        grid=(5,),
        in_specs=[pl.BlockSpec((10000, D), lambda i: (i, 0)),
                  pl.BlockSpec((10000, 1), lambda i: (i, 0))],
        out_specs=pl.BlockSpec((10000, D), lambda i: (i, 0)),
        out_shape=jax.ShapeDtypeStruct((NN, D), jnp.float32),
    )(x, v)


def _sum4_body(a_ref, b_ref, c_ref, d_ref, v_ref, o_ref):
    o_ref[...] = (a_ref[...] + b_ref[...] + c_ref[...] + d_ref[...]) * v_ref[...]


def _sum4scale(a, b, c, d, v):
    return pl.pallas_call(
        _sum4_body,
        grid=(5,),
        in_specs=[pl.BlockSpec((10000, D), lambda i: (i, 0))] * 4
        + [pl.BlockSpec((10000, 1), lambda i: (i, 0))],
        out_specs=pl.BlockSpec((10000, D), lambda i: (i, 0)),
        out_shape=jax.ShapeDtypeStruct((NN, D), jnp.float32),
    )(a, b, c, d, v)


def _loss_body(g_ref, o_ref):
    g = g_ref[...]
    u1, p1, n1 = g[0:B], g[B:2 * B], g[2 * B:3 * B]
    u2, p2, n2 = g[3 * B:4 * B], g[4 * B:5 * B], g[5 * B:6 * B]
    x = jnp.sum(u1 * (p1 - n1) + u2 * (p2 - n2), axis=1)
    ls = jnp.minimum(x, 0.0) - jnp.log1p(jnp.exp(-jnp.abs(x)))
    o_ref[...] = jnp.reshape(-jnp.mean(ls), (1, 1))


def _loss(g):
    out = pl.pallas_call(
        _loss_body,
        out_shape=jax.ShapeDtypeStruct((1, 1), jnp.float32),
    )(g)
    return out[0, 0]


def kernel(user_ids, pos_item_ids, neg_item_ids, e_u1, e_u2, e_i1, e_i2,
           adj_row, adj_col, adj_vals):
    i32 = jnp.int32
    E0 = jnp.concatenate([e_u1, e_i1], axis=0)
    E1 = jnp.concatenate([e_u2, e_i2], axis=0)

    pidx = jnp.arange(PAD, dtype=i32)
    pad_src = pidx % NN
    pad_dst = NU + pidx % (NRP - NU)
    srcp = jnp.concatenate([adj_col[:HALF].astype(i32), pad_src,
                            adj_col[HALF:].astype(i32), pad_src])
    dstp = jnp.concatenate([adj_row[:HALF].astype(i32), pad_dst,
                            adj_row[HALF:].astype(i32) - NU, pad_dst])

    deg = jnp.maximum(_deg_call(dstp), 1.0)
    s2 = lax.rsqrt(deg).reshape(NN, 1)
    q2 = (jnp.sqrt(deg) * 0.25).reshape(NN, 1)

    g0 = _rowscale(E0, s2)
    g1 = _rowscale(E1, s2)
    tabs = [(g0, g1)]
    for _ in range(3):
        tabs.append(_layer_call(tabs[-1][0], tabs[-1][1], srcp, dstp, deg))
    s0 = _sum4scale(tabs[0][0], tabs[1][0], tabs[2][0], tabs[3][0], q2)
    s1 = _sum4scale(tabs[0][1], tabs[1][1], tabs[2][1], tabs[3][1], q2)

    fidx = jnp.stack([user_ids.astype(i32),
                      NU + pos_item_ids.astype(i32),
                      NU + neg_item_ids.astype(i32)])
    fidx = fidx.reshape(3, 32, W).transpose(1, 0, 2)
    gathered = _fgather_call(s0, s1, fidx)
    return _loss(gathered)


# speedup vs baseline: 12.2795x; 1.1570x over previous
import jax
import jax.numpy as jnp
from jax import lax
from jax.experimental import pallas as pl
from jax.experimental.pallas import tpu as pltpu
import jax.experimental.pallas.tpu_sc as plsc

NU = 25000
NN = 50000
HALF = 400000
D = 64
W = 128
NWIN = 200
PER_TILE = W * NWIN
PER_SC = 16 * PER_TILE
PAD = PER_SC - HALF
STRIPE = 1568
NRP = 16 * STRIPE
NBUF = 2
WBLK = 20
NBLK = NWIN // WBLK
IBLK = WBLK * W
B = 4096


def _mesh():
    return plsc.VectorSubcoreMesh(core_axis_name="c", subcore_axis_name="s",
                                  num_cores=2, num_subcores=16)


def _sc_params():
    return pltpu.CompilerParams(use_tc_tiling_on_sc=False,
                                needs_layout_passes=False)


def _deg_body(dstp, degout, acc1, didx, ones_v, zbuf, ssem):
    c = lax.axis_index("c")
    t = lax.axis_index("s")
    vone = jnp.full((16,), 1.0, jnp.float32)
    vzero = jnp.zeros((16,), jnp.float32)

    def fill(i, carry):
        ones_v[pl.ds(i * 16, 16)] = vone
        zbuf[pl.ds(i * 16, 16)] = vzero
        return carry

    lax.fori_loop(0, W // 16, fill, 0)
    for z in range(12):
        pltpu.sync_copy(zbuf, acc1.at[pl.ds(t * STRIPE + z * W, W)])
    pltpu.sync_copy(zbuf.at[pl.ds(0, STRIPE - 12 * W)],
                    acc1.at[pl.ds(t * STRIPE + 12 * W, STRIPE - 12 * W)])
    plsc.subcore_barrier()
    ebase = c * PER_SC + t * PER_TILE

    def blk_body(blk, carry):
        pltpu.sync_copy(dstp.at[pl.ds(ebase + blk * IBLK, IBLK)], didx)

        def outer(o, carry2):
            for b in range(NBUF):
                j = o * NBUF + b
                idxw = didx.at[pl.ds(j * W, W)]
                pltpu.async_copy(ones_v, acc1.at[idxw], ssem.at[b], add=True)
                pltpu.make_async_copy(ones_v, acc1.at[idxw], ssem.at[b]).wait()
            return carry2

        return lax.fori_loop(0, WBLK // NBUF, outer, carry)

    lax.fori_loop(0, NBLK, blk_body, 0)
    plsc.subcore_barrier()

    @pl.when(t == 0)
    def _drain():
        pltpu.sync_copy(acc1.at[pl.ds(0, NU)], degout.at[pl.ds(c * NU, NU)])


def _deg_call(dstp):
    return pl.kernel(
        _deg_body,
        out_type=jax.ShapeDtypeStruct((NN,), jnp.float32),
        mesh=_mesh(),
        compiler_params=_sc_params(),
        scratch_types=[
            pltpu.VMEM_SHARED((NRP,), jnp.float32),
            pltpu.VMEM((IBLK,), jnp.int32),
            pltpu.VMEM((W,), jnp.float32),
            pltpu.VMEM((W,), jnp.float32),
            pltpu.SemaphoreType.DMA((NBUF,)),
        ],
    )(dstp)


def _layer_body(g0, g1, srcp, dstp, dinv, out0, out1,
                acc, sidx, didx, degw, rbuf, gsem, ssem):
    c = lax.axis_index("c")
    t = lax.axis_index("s")
    ebase = c * PER_SC + t * PER_TILE
    rbase = c * NU + t * STRIPE
    vzero = jnp.zeros((16,), jnp.float32)
    for chunk in range(2):
        gprev = (g0, g1)[chunk]
        gout = (out0, out1)[chunk]

        def zfill(i, carry):
            for f in range(4):
                rbuf[1, i, pl.ds(f * 16, 16)] = vzero
            return carry

        lax.fori_loop(0, W, zfill, 0)
        for z in range(12):
            pltpu.sync_copy(rbuf.at[1], acc.at[pl.ds(t * STRIPE + z * W, W)])
        pltpu.sync_copy(rbuf.at[1, pl.ds(0, STRIPE - 12 * W)],
                        acc.at[pl.ds(t * STRIPE + 12 * W, STRIPE - 12 * W)])
        plsc.subcore_barrier()

        def blk_body(blk, carry):
            pltpu.sync_copy(srcp.at[pl.ds(ebase + blk * IBLK, IBLK)], sidx)
            pltpu.sync_copy(dstp.at[pl.ds(ebase + blk * IBLK, IBLK)], didx)
            for b in range(NBUF):
                pltpu.async_copy(gprev.at[sidx.at[pl.ds(b * W, W)]],
                                 rbuf.at[b], gsem.at[b])

            def outer(o, carry2):
                for b in range(NBUF):
                    j = o * NBUF + b
                    sw = sidx.at[pl.ds(j * W, W)]
                    dw = didx.at[pl.ds(j * W, W)]
                    pltpu.make_async_copy(gprev.at[sw], rbuf.at[b],
                                          gsem.at[b]).wait()
                    pltpu.async_copy(rbuf.at[b], acc.at[dw], ssem.at[b],
                                     add=True)
                    pltpu.make_async_copy(rbuf.at[b], acc.at[dw],
                                          ssem.at[b]).wait()

                    @pl.when(j + NBUF < WBLK)
                    def _next():
                        pltpu.async_copy(
                            gprev.at[sidx.at[pl.ds((j + NBUF) * W, W)]],
                            rbuf.at[b], gsem.at[b])
                return carry2

            return lax.fori_loop(0, WBLK // NBUF, outer, carry)

        lax.fori_loop(0, NBLK, blk_body, 0)
        plsc.subcore_barrier()

        def drain_block(nrows, row0):
            pltpu.sync_copy(acc.at[pl.ds(t * STRIPE + row0, nrows)],
                            rbuf.at[0, pl.ds(0, nrows)])
            pltpu.sync_copy(dinv.at[pl.ds(rbase + row0, nrows)],
                            degw.at[pl.ds(0, nrows)])

            def rows(r, carry3):
                dv = plsc.load_gather(
                    degw, [jnp.full((16,), r, jnp.int32)])
                for f in range(4):
                    rbuf[0, r, pl.ds(f * 16, 16)] = (
                        rbuf[0, r, pl.ds(f * 16, 16)] * dv)
                return carry3

            lax.fori_loop(0, nrows, rows, 0)
            pltpu.sync_copy(rbuf.at[0, pl.ds(0, nrows)],
                            gout.at[pl.ds(rbase + row0, nrows)])

        @pl.when(t < 15)
        def _drain():
            for blkr in range(13):
                nr = W if blkr < 12 else STRIPE - 12 * W
                drain_block(nr, blkr * W)

        @pl.when(t == 15)
        def _drain_last():
            for blkr in range(12):
                nr = W if blkr < 11 else 1480 - 11 * W
                drain_block(nr, blkr * W)


def _layer_call(g0, g1, srcp, dstp, dinv):
    return pl.kernel(
        _layer_body,
        out_type=(jax.ShapeDtypeStruct((NN, D), jnp.float32),
                  jax.ShapeDtypeStruct((NN, D), jnp.float32)),
        mesh=_mesh(),
        compiler_params=_sc_params(),
        scratch_types=[
            pltpu.VMEM_SHARED((NRP, D), jnp.float32),
            pltpu.VMEM((IBLK,), jnp.int32),
            pltpu.VMEM((IBLK,), jnp.int32),
            pltpu.VMEM((W,), jnp.float32),
            pltpu.VMEM((NBUF, W, D), jnp.float32),
            pltpu.SemaphoreType.DMA((NBUF,)),
            pltpu.SemaphoreType.DMA((NBUF,)),
        ],
    )(g0, g1, srcp, dstp, dinv)


def _stripe_scale(src_ref, dst_ref, vec_ref, src_row0, dst_row0, t,
                  rbuf, vw):
    def do_block(nrows, row0):
        pltpu.sync_copy(src_ref.at[pl.ds(src_row0 + row0, nrows)],
                        rbuf.at[0, pl.ds(0, nrows)])
        pltpu.sync_copy(vec_ref.at[pl.ds(dst_row0 + row0, nrows)],
                        vw.at[pl.ds(0, nrows)])

        def rows(r, carry):
            sv = plsc.load_gather(vw, [jnp.full((16,), r, jnp.int32)])
            for f in range(4):
                rbuf[0, r, pl.ds(f * 16, 16)] = (
                    rbuf[0, r, pl.ds(f * 16, 16)] * sv)
            return carry

        lax.fori_loop(0, nrows, rows, 0)
        pltpu.sync_copy(rbuf.at[0, pl.ds(0, nrows)],
                        dst_ref.at[pl.ds(dst_row0 + row0, nrows)])

    @pl.when(t < 15)
    def _go():
        for blkr in range(13):
            nr = W if blkr < 12 else STRIPE - 12 * W
            do_block(nr, blkr * W)

    @pl.when(t == 15)
    def _go_last():
        for blkr in range(12):
            nr = W if blkr < 11 else 1480 - 11 * W
            do_block(nr, blkr * W)


def _scale0_body(eu1, ei1, eu2, ei2, s_all, g0, g1, rbuf, vw):
    c = lax.axis_index("c")
    t = lax.axis_index("s")

    @pl.when(c == 0)
    def _c0():
        _stripe_scale(eu1, g0, s_all, t * STRIPE, t * STRIPE, t, rbuf, vw)
        _stripe_scale(ei1, g0, s_all, t * STRIPE, NU + t * STRIPE, t, rbuf, vw)

    @pl.when(c == 1)
    def _c1():
        _stripe_scale(eu2, g1, s_all, t * STRIPE, t * STRIPE, t, rbuf, vw)
        _stripe_scale(ei2, g1, s_all, t * STRIPE, NU + t * STRIPE, t, rbuf, vw)


def _scale0_call(eu1, ei1, eu2, ei2, s_all):
    return pl.kernel(
        _scale0_body,
        out_type=(jax.ShapeDtypeStruct((NN, D), jnp.float32),
                  jax.ShapeDtypeStruct((NN, D), jnp.float32)),
        mesh=_mesh(),
        compiler_params=_sc_params(),
        scratch_types=[
            pltpu.VMEM((1, W, D), jnp.float32),
            pltpu.VMEM((W,), jnp.float32),
        ],
    )(eu1, ei1, eu2, ei2, s_all)


def _fgather_body(a0, b0, c0, d0, a1, b1, c1, d1, q_all, fidx, out,
                  fidx_v, gb, ob, qw, gsem):
    c = lax.axis_index("c")
    t = lax.axis_index("s")
    wid = c * 16 + t
    pltpu.sync_copy(fidx.at[wid], fidx_v)
    for chunk in range(2):
        tabs = (a0, b0, c0, d0) if chunk == 0 else (a1, b1, c1, d1)
        for role in range(3):
            idxw = fidx_v.at[role]
            for k in range(4):
                pltpu.async_copy(tabs[k].at[idxw], gb.at[k], gsem.at[k])
            pltpu.sync_copy(q_all.at[idxw], qw)
            for k in range(4):
                pltpu.make_async_copy(tabs[k].at[idxw], gb.at[k],
                                      gsem.at[k]).wait()

            def rows(r, carry):
                qv = plsc.load_gather(qw, [jnp.full((16,), r, jnp.int32)])
                for f in range(4):
                    sl = pl.ds(f * 16, 16)
                    ob[r, sl] = ((gb[0, r, sl] + gb[1, r, sl])
                                 + (gb[2, r, sl] + gb[3, r, sl])) * qv
                return carry

            lax.fori_loop(0, W, rows, 0)
            obase = (chunk * 3 + role) * B + wid * W
            pltpu.sync_copy(ob, out.at[pl.ds(obase, W)])


def _fgather_call(t0, t1, t2, t3, q_all, fidx):
    return pl.kernel(
        _fgather_body,
        out_type=jax.ShapeDtypeStruct((6 * B, D), jnp.float32),
        mesh=_mesh(),
        compiler_params=_sc_params(),
        scratch_types=[
            pltpu.VMEM((3, W), jnp.int32),
            pltpu.VMEM((4, W, D), jnp.float32),
            pltpu.VMEM((W, D), jnp.float32),
            pltpu.VMEM((W,), jnp.float32),
            pltpu.SemaphoreType.DMA((4,)),
        ],
    )(t0[0], t1[0], t2[0], t3[0], t0[1], t1[1], t2[1], t3[1], q_all, fidx)


def _rowscale_body(x_ref, v_ref, o_ref):
    o_ref[...] = x_ref[...] * v_ref[...]


def _rowscale(x, v):
    return pl.pallas_call(
        _rowscale_body,
        grid=(5,),
        in_specs=[pl.BlockSpec((10000, D), lambda i: (i, 0)),
                  pl.BlockSpec((10000, 1), lambda i: (i, 0))],
        out_specs=pl.BlockSpec((10000, D), lambda i: (i, 0)),
        out_shape=jax.ShapeDtypeStruct((NN, D), jnp.float32),
    )(x, v)


def _sum4_body(a_ref, b_ref, c_ref, d_ref, v_ref, o_ref):
    o_ref[...] = (a_ref[...] + b_ref[...] + c_ref[...] + d_ref[...]) * v_ref[...]


def _sum4scale(a, b, c, d, v):
    return pl.pallas_call(
        _sum4_body,
        grid=(5,),
        in_specs=[pl.BlockSpec((10000, D), lambda i: (i, 0))] * 4
        + [pl.BlockSpec((10000, 1), lambda i: (i, 0))],
        out_specs=pl.BlockSpec((10000, D), lambda i: (i, 0)),
        out_shape=jax.ShapeDtypeStruct((NN, D), jnp.float32),
    )(a, b, c, d, v)


def _loss_body(g_ref, o_ref):
    g = g_ref[...]
    u1, p1, n1 = g[0:B], g[B:2 * B], g[2 * B:3 * B]
    u2, p2, n2 = g[3 * B:4 * B], g[4 * B:5 * B], g[5 * B:6 * B]
    x = jnp.sum(u1 * (p1 - n1) + u2 * (p2 - n2), axis=1)
    ls = jnp.minimum(x, 0.0) - jnp.log1p(jnp.exp(-jnp.abs(x)))
    o_ref[...] = jnp.reshape(-jnp.mean(ls), (1, 1))


def _loss(g):
    out = pl.pallas_call(
        _loss_body,
        out_shape=jax.ShapeDtypeStruct((1, 1), jnp.float32),
    )(g)
    return out[0, 0]


def kernel(user_ids, pos_item_ids, neg_item_ids, e_u1, e_u2, e_i1, e_i2,
           adj_row, adj_col, adj_vals):
    i32 = jnp.int32

    pidx = jnp.arange(PAD, dtype=i32)
    pad_src = pidx % NN
    pad_dst = NU + pidx % (NRP - NU)
    srcp = jnp.concatenate([adj_col[:HALF].astype(i32), pad_src,
                            adj_col[HALF:].astype(i32), pad_src])
    dstp = jnp.concatenate([adj_row[:HALF].astype(i32), pad_dst,
                            adj_row[HALF:].astype(i32) - NU, pad_dst])

    deg = jnp.maximum(_deg_call(dstp), 1.0)
    s_all = lax.rsqrt(deg)
    dinv = 1.0 / deg
    q_all = jnp.sqrt(deg) * 0.25

    tabs = [_scale0_call(e_u1, e_i1, e_u2, e_i2, s_all)]
    for _ in range(3):
        tabs.append(_layer_call(tabs[-1][0], tabs[-1][1], srcp, dstp, dinv))

    fidx = jnp.stack([user_ids.astype(i32),
                      NU + pos_item_ids.astype(i32),
                      NU + neg_item_ids.astype(i32)])
    fidx = fidx.reshape(3, 32, W).transpose(1, 0, 2)
    gathered = _fgather_call(tabs[0], tabs[1], tabs[2], tabs[3],
                             q_all, fidx)
    return _loss(gathered)


# speedup vs baseline: 12.8837x; 1.0492x over previous
import jax
import jax.numpy as jnp
from jax import lax
from jax.experimental import pallas as pl
from jax.experimental.pallas import tpu as pltpu
import jax.experimental.pallas.tpu_sc as plsc

NU = 25000
NN = 50000
HALF = 400000
D = 64
W = 128
NWIN = 204
PER_TILE = W * NWIN
PER_SC = 16 * PER_TILE
PAD = PER_SC - HALF
STRIPE = 1568
NRP = 16 * STRIPE
NBUF = 3
WBLK = 12
NBLK = NWIN // WBLK
IBLK = WBLK * W
B = 4096


def _mesh():
    return plsc.VectorSubcoreMesh(core_axis_name="c", subcore_axis_name="s",
                                  num_cores=2, num_subcores=16)


def _sc_params():
    return pltpu.CompilerParams(use_tc_tiling_on_sc=False,
                                needs_layout_passes=False)


def _deg_body(dstp, degout, acc1, didx, ones_v, zbuf, ssem):
    c = lax.axis_index("c")
    t = lax.axis_index("s")
    vone = jnp.full((16,), 1.0, jnp.float32)
    vzero = jnp.zeros((16,), jnp.float32)

    def fill(i, carry):
        ones_v[pl.ds(i * 16, 16)] = vone
        zbuf[pl.ds(i * 16, 16)] = vzero
        return carry

    lax.fori_loop(0, W // 16, fill, 0)
    for z in range(12):
        pltpu.sync_copy(zbuf, acc1.at[pl.ds(t * STRIPE + z * W, W)])
    pltpu.sync_copy(zbuf.at[pl.ds(0, STRIPE - 12 * W)],
                    acc1.at[pl.ds(t * STRIPE + 12 * W, STRIPE - 12 * W)])
    plsc.subcore_barrier()
    ebase = c * PER_SC + t * PER_TILE

    def blk_body(blk, carry):
        pltpu.sync_copy(dstp.at[pl.ds(ebase + blk * IBLK, IBLK)], didx)

        def outer(o, carry2):
            for b in range(NBUF):
                j = o * NBUF + b
                idxw = didx.at[pl.ds(j * W, W)]
                pltpu.async_copy(ones_v, acc1.at[idxw], ssem.at[b], add=True)
                pltpu.make_async_copy(ones_v, acc1.at[idxw], ssem.at[b]).wait()
            return carry2

        return lax.fori_loop(0, WBLK // NBUF, outer, carry)

    lax.fori_loop(0, NBLK, blk_body, 0)
    plsc.subcore_barrier()

    @pl.when(t == 0)
    def _drain():
        pltpu.sync_copy(acc1.at[pl.ds(0, NU)], degout.at[pl.ds(c * NU, NU)])


def _deg_call(dstp):
    return pl.kernel(
        _deg_body,
        out_type=jax.ShapeDtypeStruct((NN,), jnp.float32),
        mesh=_mesh(),
        compiler_params=_sc_params(),
        scratch_types=[
            pltpu.VMEM_SHARED((NRP,), jnp.float32),
            pltpu.VMEM((IBLK,), jnp.int32),
            pltpu.VMEM((W,), jnp.float32),
            pltpu.VMEM((W,), jnp.float32),
            pltpu.SemaphoreType.DMA((NBUF,)),
        ],
    )(dstp)


def _layer_body(g0, g1, srcp, dstp, dinv, out0, out1,
                acc, sidx, didx, degw, rbuf, gsem, ssem):
    c = lax.axis_index("c")
    t = lax.axis_index("s")
    ebase = c * PER_SC + t * PER_TILE
    rbase = c * NU + t * STRIPE
    vzero = jnp.zeros((16,), jnp.float32)
    for chunk in range(2):
        gprev = (g0, g1)[chunk]
        gout = (out0, out1)[chunk]

        def zfill(i, carry):
            for f in range(4):
                rbuf[1, i, pl.ds(f * 16, 16)] = vzero
            return carry

        lax.fori_loop(0, W, zfill, 0)
        for z in range(12):
            pltpu.sync_copy(rbuf.at[1], acc.at[pl.ds(t * STRIPE + z * W, W)])
        pltpu.sync_copy(rbuf.at[1, pl.ds(0, STRIPE - 12 * W)],
                        acc.at[pl.ds(t * STRIPE + 12 * W, STRIPE - 12 * W)])
        plsc.subcore_barrier()

        def blk_body(blk, carry):
            pltpu.sync_copy(srcp.at[pl.ds(ebase + blk * IBLK, IBLK)], sidx)
            pltpu.sync_copy(dstp.at[pl.ds(ebase + blk * IBLK, IBLK)], didx)
            for b in range(NBUF):
                pltpu.async_copy(gprev.at[sidx.at[pl.ds(b * W, W)]],
                                 rbuf.at[b], gsem.at[b])

            def outer(o, carry2):
                for b in range(NBUF):
                    j = o * NBUF + b
                    sw = sidx.at[pl.ds(j * W, W)]
                    dw = didx.at[pl.ds(j * W, W)]
                    pltpu.make_async_copy(gprev.at[sw], rbuf.at[b],
                                          gsem.at[b]).wait()
                    pltpu.async_copy(rbuf.at[b], acc.at[dw], ssem.at[b],
                                     add=True)

                    @pl.when(j + NBUF < WBLK)
                    def _next():
                        pltpu.make_async_copy(rbuf.at[b], acc.at[dw],
                                              ssem.at[b]).wait()
                        pltpu.async_copy(
                            gprev.at[sidx.at[pl.ds((j + NBUF) * W, W)]],
                            rbuf.at[b], gsem.at[b])
                return carry2

            carry = lax.fori_loop(0, WBLK // NBUF, outer, carry)
            for b in range(NBUF):
                dw = didx.at[pl.ds((WBLK - NBUF + b) * W, W)]
                pltpu.make_async_copy(rbuf.at[b], acc.at[dw],
                                      ssem.at[b]).wait()
            return carry

        lax.fori_loop(0, NBLK, blk_body, 0)
        plsc.subcore_barrier()

        def drain_block(nrows, row0):
            pltpu.sync_copy(acc.at[pl.ds(t * STRIPE + row0, nrows)],
                            rbuf.at[0, pl.ds(0, nrows)])
            pltpu.sync_copy(dinv.at[pl.ds(rbase + row0, nrows)],
                            degw.at[pl.ds(0, nrows)])

            def rows(r, carry3):
                dv = plsc.load_gather(
                    degw, [jnp.full((16,), r, jnp.int32)])
                for f in range(4):
                    rbuf[0, r, pl.ds(f * 16, 16)] = (
                        rbuf[0, r, pl.ds(f * 16, 16)] * dv)
                return carry3

            lax.fori_loop(0, nrows, rows, 0)
            pltpu.sync_copy(rbuf.at[0, pl.ds(0, nrows)],
                            gout.at[pl.ds(rbase + row0, nrows)])

        @pl.when(t < 15)
        def _drain():
            for blkr in range(13):
                nr = W if blkr < 12 else STRIPE - 12 * W
                drain_block(nr, blkr * W)

        @pl.when(t == 15)
        def _drain_last():
            for blkr in range(12):
                nr = W if blkr < 11 else 1480 - 11 * W
                drain_block(nr, blkr * W)


def _layer_call(g0, g1, srcp, dstp, dinv):
    return pl.kernel(
        _layer_body,
        out_type=(jax.ShapeDtypeStruct((NN, D), jnp.float32),
                  jax.ShapeDtypeStruct((NN, D), jnp.float32)),
        mesh=_mesh(),
        compiler_params=_sc_params(),
        scratch_types=[
            pltpu.VMEM_SHARED((NRP, D), jnp.float32),
            pltpu.VMEM((IBLK,), jnp.int32),
            pltpu.VMEM((IBLK,), jnp.int32),
            pltpu.VMEM((W,), jnp.float32),
            pltpu.VMEM((NBUF, W, D), jnp.float32),
            pltpu.SemaphoreType.DMA((NBUF,)),
            pltpu.SemaphoreType.DMA((NBUF,)),
        ],
    )(g0, g1, srcp, dstp, dinv)


def _stripe_scale(src_ref, dst_ref, vec_ref, src_row0, dst_row0, t,
                  rbuf, vw):
    def do_block(nrows, row0):
        pltpu.sync_copy(src_ref.at[pl.ds(src_row0 + row0, nrows)],
                        rbuf.at[0, pl.ds(0, nrows)])
        pltpu.sync_copy(vec_ref.at[pl.ds(dst_row0 + row0, nrows)],
                        vw.at[pl.ds(0, nrows)])

        def rows(r, carry):
            sv = plsc.load_gather(vw, [jnp.full((16,), r, jnp.int32)])
            for f in range(4):
                rbuf[0, r, pl.ds(f * 16, 16)] = (
                    rbuf[0, r, pl.ds(f * 16, 16)] * sv)
            return carry

        lax.fori_loop(0, nrows, rows, 0)
        pltpu.sync_copy(rbuf.at[0, pl.ds(0, nrows)],
                        dst_ref.at[pl.ds(dst_row0 + row0, nrows)])

    @pl.when(t < 15)
    def _go():
        for blkr in range(13):
            nr = W if blkr < 12 else STRIPE - 12 * W
            do_block(nr, blkr * W)

    @pl.when(t == 15)
    def _go_last():
        for blkr in range(12):
            nr = W if blkr < 11 else 1480 - 11 * W
            do_block(nr, blkr * W)


def _scale0_body(eu1, ei1, eu2, ei2, s_all, g0, g1, rbuf, vw):
    c = lax.axis_index("c")
    t = lax.axis_index("s")

    @pl.when(c == 0)
    def _c0():
        _stripe_scale(eu1, g0, s_all, t * STRIPE, t * STRIPE, t, rbuf, vw)
        _stripe_scale(ei1, g0, s_all, t * STRIPE, NU + t * STRIPE, t, rbuf, vw)

    @pl.when(c == 1)
    def _c1():
        _stripe_scale(eu2, g1, s_all, t * STRIPE, t * STRIPE, t, rbuf, vw)
        _stripe_scale(ei2, g1, s_all, t * STRIPE, NU + t * STRIPE, t, rbuf, vw)


def _scale0_call(eu1, ei1, eu2, ei2, s_all):
    return pl.kernel(
        _scale0_body,
        out_type=(jax.ShapeDtypeStruct((NN, D), jnp.float32),
                  jax.ShapeDtypeStruct((NN, D), jnp.float32)),
        mesh=_mesh(),
        compiler_params=_sc_params(),
        scratch_types=[
            pltpu.VMEM((1, W, D), jnp.float32),
            pltpu.VMEM((W,), jnp.float32),
        ],
    )(eu1, ei1, eu2, ei2, s_all)


def _fgather_body(a0, b0, c0, d0, a1, b1, c1, d1, q_all, fidx, out,
                  fidx_v, gb, ob, qw, gsem):
    c = lax.axis_index("c")
    t = lax.axis_index("s")
    wid = c * 16 + t
    pltpu.sync_copy(fidx.at[wid], fidx_v)
    for chunk in range(2):
        tabs = (a0, b0, c0, d0) if chunk == 0 else (a1, b1, c1, d1)
        for role in range(3):
            idxw = fidx_v.at[role]
            for k in range(4):
                pltpu.async_copy(tabs[k].at[idxw], gb.at[k], gsem.at[k])
            pltpu.sync_copy(q_all.at[idxw], qw)
            for k in range(4):
                pltpu.make_async_copy(tabs[k].at[idxw], gb.at[k],
                                      gsem.at[k]).wait()

            def rows(r, carry):
                qv = plsc.load_gather(qw, [jnp.full((16,), r, jnp.int32)])
                for f in range(4):
                    sl = pl.ds(f * 16, 16)
                    ob[r, sl] = ((gb[0, r, sl] + gb[1, r, sl])
                                 + (gb[2, r, sl] + gb[3, r, sl])) * qv
                return carry

            lax.fori_loop(0, W, rows, 0)
            obase = (chunk * 3 + role) * B + wid * W
            pltpu.sync_copy(ob, out.at[pl.ds(obase, W)])


def _fgather_call(t0, t1, t2, t3, q_all, fidx):
    return pl.kernel(
        _fgather_body,
        out_type=jax.ShapeDtypeStruct((6 * B, D), jnp.float32),
        mesh=_mesh(),
        compiler_params=_sc_params(),
        scratch_types=[
            pltpu.VMEM((3, W), jnp.int32),
            pltpu.VMEM((4, W, D), jnp.float32),
            pltpu.VMEM((W, D), jnp.float32),
            pltpu.VMEM((W,), jnp.float32),
            pltpu.SemaphoreType.DMA((4,)),
        ],
    )(t0[0], t1[0], t2[0], t3[0], t0[1], t1[1], t2[1], t3[1], q_all, fidx)


def _rowscale_body(x_ref, v_ref, o_ref):
    o_ref[...] = x_ref[...] * v_ref[...]


def _rowscale(x, v):
    return pl.pallas_call(
        _rowscale_body,
        grid=(5,),
        in_specs=[pl.BlockSpec((10000, D), lambda i: (i, 0)),
                  pl.BlockSpec((10000, 1), lambda i: (i, 0))],
        out_specs=pl.BlockSpec((10000, D), lambda i: (i, 0)),
        out_shape=jax.ShapeDtypeStruct((NN, D), jnp.float32),
    )(x, v)


def _sum4_body(a_ref, b_ref, c_ref, d_ref, v_ref, o_ref):
    o_ref[...] = (a_ref[...] + b_ref[...] + c_ref[...] + d_ref[...]) * v_ref[...]


def _sum4scale(a, b, c, d, v):
    return pl.pallas_call(
        _sum4_body,
        grid=(5,),
        in_specs=[pl.BlockSpec((10000, D), lambda i: (i, 0))] * 4
        + [pl.BlockSpec((10000, 1), lambda i: (i, 0))],
        out_specs=pl.BlockSpec((10000, D), lambda i: (i, 0)),
        out_shape=jax.ShapeDtypeStruct((NN, D), jnp.float32),
    )(a, b, c, d, v)


def _loss_body(g_ref, o_ref):
    g = g_ref[...]
    u1, p1, n1 = g[0:B], g[B:2 * B], g[2 * B:3 * B]
    u2, p2, n2 = g[3 * B:4 * B], g[4 * B:5 * B], g[5 * B:6 * B]
    x = jnp.sum(u1 * (p1 - n1) + u2 * (p2 - n2), axis=1)
    ls = jnp.minimum(x, 0.0) - jnp.log1p(jnp.exp(-jnp.abs(x)))
    o_ref[...] = jnp.reshape(-jnp.mean(ls), (1, 1))


def _loss(g):
    out = pl.pallas_call(
        _loss_body,
        out_shape=jax.ShapeDtypeStruct((1, 1), jnp.float32),
    )(g)
    return out[0, 0]


def kernel(user_ids, pos_item_ids, neg_item_ids, e_u1, e_u2, e_i1, e_i2,
           adj_row, adj_col, adj_vals):
    i32 = jnp.int32

    pidx = jnp.arange(PAD, dtype=i32)
    pad_src = pidx % NN
    pad_dst = NU + pidx % (NRP - NU)
    srcp = jnp.concatenate([adj_col[:HALF].astype(i32), pad_src,
                            adj_col[HALF:].astype(i32), pad_src])
    dstp = jnp.concatenate([adj_row[:HALF].astype(i32), pad_dst,
                            adj_row[HALF:].astype(i32) - NU, pad_dst])

    deg = jnp.maximum(_deg_call(dstp), 1.0)
    s_all = lax.rsqrt(deg)
    dinv = 1.0 / deg
    q_all = jnp.sqrt(deg) * 0.25

    tabs = [_scale0_call(e_u1, e_i1, e_u2, e_i2, s_all)]
    for _ in range(3):
        tabs.append(_layer_call(tabs[-1][0], tabs[-1][1], srcp, dstp, dinv))

    fidx = jnp.stack([user_ids.astype(i32),
                      NU + pos_item_ids.astype(i32),
                      NU + neg_item_ids.astype(i32)])
    fidx = fidx.reshape(3, 32, W).transpose(1, 0, 2)
    gathered = _fgather_call(tabs[0], tabs[1], tabs[2], tabs[3],
                             q_all, fidx)
    return _loss(gathered)


# speedup vs baseline: 13.6800x; 1.0618x over previous
import jax
import jax.numpy as jnp
from jax import lax
from jax.experimental import pallas as pl
from jax.experimental.pallas import tpu as pltpu
import jax.experimental.pallas.tpu_sc as plsc

NU = 25000
NN = 50000
HALF = 400000
D = 64
W = 128
NWIN = 195
NWSC = HALF // W
STRIPE = 1568
NRP = 16 * STRIPE
NBUF = 3
WBLK = 15
NBLK = NWIN // WBLK
IBLK = WBLK * W
B = 4096


def _mesh():
    return plsc.VectorSubcoreMesh(core_axis_name="c", subcore_axis_name="s",
                                  num_cores=2, num_subcores=16)


def _sc_params():
    return pltpu.CompilerParams(use_tc_tiling_on_sc=False,
                                needs_layout_passes=False)


def _deg_body(dstp, degout, acc1, didx, ones_v, zbuf, ssem):
    c = lax.axis_index("c")
    t = lax.axis_index("s")
    vone = jnp.full((16,), 1.0, jnp.float32)
    vzero = jnp.zeros((16,), jnp.float32)

    def fill(i, carry):
        ones_v[pl.ds(i * 16, 16)] = vone
        zbuf[pl.ds(i * 16, 16)] = vzero
        return carry

    lax.fori_loop(0, W // 16, fill, 0)
    for z in range(12):
        pltpu.sync_copy(zbuf, acc1.at[pl.ds(t * STRIPE + z * W, W)])
    pltpu.sync_copy(zbuf.at[pl.ds(0, STRIPE - 12 * W)],
                    acc1.at[pl.ds(t * STRIPE + 12 * W, STRIPE - 12 * W)])
    plsc.subcore_barrier()
    ebase = c * HALF + t * NWIN * W

    def blk_body(blk, carry):
        pltpu.sync_copy(dstp.at[pl.ds(ebase + blk * IBLK, IBLK)], didx)

        def outer(o, carry2):
            for b in range(NBUF):
                j = o * NBUF + b
                idxw = didx.at[pl.ds(j * W, W)]
                pltpu.async_copy(ones_v, acc1.at[idxw], ssem.at[b], add=True)
                pltpu.make_async_copy(ones_v, acc1.at[idxw], ssem.at[b]).wait()
            return carry2

        return lax.fori_loop(0, WBLK // NBUF, outer, carry)

    lax.fori_loop(0, NBLK, blk_body, 0)

    @pl.when(t == 15)
    def _tail():
        pltpu.sync_copy(dstp.at[pl.ds(c * HALF + NWSC * W - 5 * W, 5 * W)],
                        didx.at[pl.ds(0, 5 * W)])
        for j in range(5):
            idxw = didx.at[pl.ds(j * W, W)]
            pltpu.sync_copy(ones_v, acc1.at[idxw], add=True)

    plsc.subcore_barrier()

    @pl.when(t == 0)
    def _drain():
        pltpu.sync_copy(acc1.at[pl.ds(0, NU)], degout.at[pl.ds(c * NU, NU)])


def _deg_call(dstp):
    return pl.kernel(
        _deg_body,
        out_type=jax.ShapeDtypeStruct((NN,), jnp.float32),
        mesh=_mesh(),
        compiler_params=_sc_params(),
        scratch_types=[
            pltpu.VMEM_SHARED((NRP,), jnp.float32),
            pltpu.VMEM((IBLK,), jnp.int32),
            pltpu.VMEM((W,), jnp.float32),
            pltpu.VMEM((W,), jnp.float32),
            pltpu.SemaphoreType.DMA((NBUF,)),
        ],
    )(dstp)


def _layer_body(g0, g1, srcp, dstp, dinv, out0, out1,
                acc, sidx, didx, degw, rbuf, gsem, ssem):
    c = lax.axis_index("c")
    t = lax.axis_index("s")
    ebase = c * HALF + t * NWIN * W
    rbase = c * NU + t * STRIPE
    vzero = jnp.zeros((16,), jnp.float32)
    for chunk in range(2):
        gprev = (g0, g1)[chunk]
        gout = (out0, out1)[chunk]

        def zfill(i, carry):
            for f in range(4):
                rbuf[1, i, pl.ds(f * 16, 16)] = vzero
            return carry

        lax.fori_loop(0, W, zfill, 0)
        for z in range(12):
            pltpu.sync_copy(rbuf.at[1], acc.at[pl.ds(t * STRIPE + z * W, W)])
        pltpu.sync_copy(rbuf.at[1, pl.ds(0, STRIPE - 12 * W)],
                        acc.at[pl.ds(t * STRIPE + 12 * W, STRIPE - 12 * W)])
        plsc.subcore_barrier()

        def blk_body(blk, carry):
            pltpu.sync_copy(srcp.at[pl.ds(ebase + blk * IBLK, IBLK)], sidx)
            pltpu.sync_copy(dstp.at[pl.ds(ebase + blk * IBLK, IBLK)], didx)
            for b in range(NBUF):
                pltpu.async_copy(gprev.at[sidx.at[pl.ds(b * W, W)]],
                                 rbuf.at[b], gsem.at[b])

            def outer(o, carry2):
                for b in range(NBUF):
                    j = o * NBUF + b
                    sw = sidx.at[pl.ds(j * W, W)]
                    dw = didx.at[pl.ds(j * W, W)]
                    pltpu.make_async_copy(gprev.at[sw], rbuf.at[b],
                                          gsem.at[b]).wait()
                    pltpu.async_copy(rbuf.at[b], acc.at[dw], ssem.at[b],
                                     add=True)

                    @pl.when(j + NBUF < WBLK)
                    def _next():
                        pltpu.make_async_copy(rbuf.at[b], acc.at[dw],
                                              ssem.at[b]).wait()
                        pltpu.async_copy(
                            gprev.at[sidx.at[pl.ds((j + NBUF) * W, W)]],
                            rbuf.at[b], gsem.at[b])
                return carry2

            carry = lax.fori_loop(0, WBLK // NBUF, outer, carry)
            for b in range(NBUF):
                dw = didx.at[pl.ds((WBLK - NBUF + b) * W, W)]
                pltpu.make_async_copy(rbuf.at[b], acc.at[dw],
                                      ssem.at[b]).wait()
            return carry

        lax.fori_loop(0, NBLK, blk_body, 0)

        @pl.when(t == 15)
        def _tail():
            pltpu.sync_copy(
                srcp.at[pl.ds(c * HALF + NWSC * W - 5 * W, 5 * W)],
                sidx.at[pl.ds(0, 5 * W)])
            pltpu.sync_copy(
                dstp.at[pl.ds(c * HALF + NWSC * W - 5 * W, 5 * W)],
                didx.at[pl.ds(0, 5 * W)])
            for j in range(5):
                sw = sidx.at[pl.ds(j * W, W)]
                dw = didx.at[pl.ds(j * W, W)]
                pltpu.sync_copy(gprev.at[sw], rbuf.at[0])
                pltpu.sync_copy(rbuf.at[0], acc.at[dw], add=True)

        plsc.subcore_barrier()

        def drain_block(nrows, row0):
            pltpu.sync_copy(acc.at[pl.ds(t * STRIPE + row0, nrows)],
                            rbuf.at[0, pl.ds(0, nrows)])
            pltpu.sync_copy(dinv.at[pl.ds(rbase + row0, nrows)],
                            degw.at[pl.ds(0, nrows)])

            def rows(r, carry3):
                dv = plsc.load_gather(
                    degw, [jnp.full((16,), r, jnp.int32)])
                for f in range(4):
                    rbuf[0, r, pl.ds(f * 16, 16)] = (
                        rbuf[0, r, pl.ds(f * 16, 16)] * dv)
                return carry3

            lax.fori_loop(0, nrows, rows, 0)
            pltpu.sync_copy(rbuf.at[0, pl.ds(0, nrows)],
                            gout.at[pl.ds(rbase + row0, nrows)])

        @pl.when(t < 15)
        def _drain():
            for blkr in range(13):
                nr = W if blkr < 12 else STRIPE - 12 * W
                drain_block(nr, blkr * W)

        @pl.when(t == 15)
        def _drain_last():
            for blkr in range(12):
                nr = W if blkr < 11 else 1480 - 11 * W
                drain_block(nr, blkr * W)


def _layer_call(g0, g1, srcp, dstp, dinv):
    return pl.kernel(
        _layer_body,
        out_type=(jax.ShapeDtypeStruct((NN, D), jnp.float32),
                  jax.ShapeDtypeStruct((NN, D), jnp.float32)),
        mesh=_mesh(),
        compiler_params=_sc_params(),
        scratch_types=[
            pltpu.VMEM_SHARED((NRP, D), jnp.float32),
            pltpu.VMEM((IBLK,), jnp.int32),
            pltpu.VMEM((IBLK,), jnp.int32),
            pltpu.VMEM((W,), jnp.float32),
            pltpu.VMEM((NBUF, W, D), jnp.float32),
            pltpu.SemaphoreType.DMA((NBUF,)),
            pltpu.SemaphoreType.DMA((NBUF,)),
        ],
    )(g0, g1, srcp, dstp, dinv)


def _stripe_scale(src_ref, dst_ref, vec_ref, src_row0, dst_row0, t,
                  rbuf, vw):
    def do_block(nrows, row0):
        pltpu.sync_copy(src_ref.at[pl.ds(src_row0 + row0, nrows)],
                        rbuf.at[0, pl.ds(0, nrows)])
        pltpu.sync_copy(vec_ref.at[pl.ds(dst_row0 + row0, nrows)],
                        vw.at[pl.ds(0, nrows)])

        def rows(r, carry):
            sv = plsc.load_gather(vw, [jnp.full((16,), r, jnp.int32)])
            for f in range(4):
                rbuf[0, r, pl.ds(f * 16, 16)] = (
                    rbuf[0, r, pl.ds(f * 16, 16)] * sv)
            return carry

        lax.fori_loop(0, nrows, rows, 0)
        pltpu.sync_copy(rbuf.at[0, pl.ds(0, nrows)],
                        dst_ref.at[pl.ds(dst_row0 + row0, nrows)])

    @pl.when(t < 15)
    def _go():
        for blkr in range(13):
            nr = W if blkr < 12 else STRIPE - 12 * W
            do_block(nr, blkr * W)

    @pl.when(t == 15)
    def _go_last():
        for blkr in range(12):
            nr = W if blkr < 11 else 1480 - 11 * W
            do_block(nr, blkr * W)


def _scale0_body(eu1, ei1, eu2, ei2, s_all, g0, g1, rbuf, vw):
    c = lax.axis_index("c")
    t = lax.axis_index("s")

    @pl.when(c == 0)
    def _c0():
        _stripe_scale(eu1, g0, s_all, t * STRIPE, t * STRIPE, t, rbuf, vw)
        _stripe_scale(ei1, g0, s_all, t * STRIPE, NU + t * STRIPE, t, rbuf, vw)

    @pl.when(c == 1)
    def _c1():
        _stripe_scale(eu2, g1, s_all, t * STRIPE, t * STRIPE, t, rbuf, vw)
        _stripe_scale(ei2, g1, s_all, t * STRIPE, NU + t * STRIPE, t, rbuf, vw)


def _scale0_call(eu1, ei1, eu2, ei2, s_all):
    return pl.kernel(
        _scale0_body,
        out_type=(jax.ShapeDtypeStruct((NN, D), jnp.float32),
                  jax.ShapeDtypeStruct((NN, D), jnp.float32)),
        mesh=_mesh(),
        compiler_params=_sc_params(),
        scratch_types=[
            pltpu.VMEM((1, W, D), jnp.float32),
            pltpu.VMEM((W,), jnp.float32),
        ],
    )(eu1, ei1, eu2, ei2, s_all)


def _fgather_body(a0, b0, c0, d0, a1, b1, c1, d1, q_all, uid, pid, nid, out,
                  fidx_v, gb, ob, qw, gsem):
    c = lax.axis_index("c")
    t = lax.axis_index("s")
    wid = c * 16 + t
    for role, ids in enumerate((uid, pid, nid)):
        pltpu.sync_copy(ids.at[pl.ds(wid * W, W)], fidx_v.at[role])
    for chunk in range(2):
        tabs = (a0, b0, c0, d0) if chunk == 0 else (a1, b1, c1, d1)
        for role in range(3):
            idxw = fidx_v.at[role]
            for k in range(4):
                pltpu.async_copy(tabs[k].at[idxw], gb.at[k], gsem.at[k])
            pltpu.sync_copy(q_all.at[idxw], qw)
            for k in range(4):
                pltpu.make_async_copy(tabs[k].at[idxw], gb.at[k],
                                      gsem.at[k]).wait()

            def rows(r, carry):
                qv = plsc.load_gather(qw, [jnp.full((16,), r, jnp.int32)])
                for f in range(4):
                    sl = pl.ds(f * 16, 16)
                    ob[r, sl] = ((gb[0, r, sl] + gb[1, r, sl])
                                 + (gb[2, r, sl] + gb[3, r, sl])) * qv
                return carry

            lax.fori_loop(0, W, rows, 0)
            obase = (chunk * 3 + role) * B + wid * W
            pltpu.sync_copy(ob, out.at[pl.ds(obase, W)])


def _fgather_call(t0, t1, t2, t3, q_all, uid, pid, nid):
    return pl.kernel(
        _fgather_body,
        out_type=jax.ShapeDtypeStruct((6 * B, D), jnp.float32),
        mesh=_mesh(),
        compiler_params=_sc_params(),
        scratch_types=[
            pltpu.VMEM((3, W), jnp.int32),
            pltpu.VMEM((4, W, D), jnp.float32),
            pltpu.VMEM((W, D), jnp.float32),
            pltpu.VMEM((W,), jnp.float32),
            pltpu.SemaphoreType.DMA((4,)),
        ],
    )(t0[0], t1[0], t2[0], t3[0], t0[1], t1[1], t2[1], t3[1], q_all,
      uid, pid, nid)


def _rowscale_body(x_ref, v_ref, o_ref):
    o_ref[...] = x_ref[...] * v_ref[...]


def _rowscale(x, v):
    return pl.pallas_call(
        _rowscale_body,
        grid=(5,),
        in_specs=[pl.BlockSpec((10000, D), lambda i: (i, 0)),
                  pl.BlockSpec((10000, 1), lambda i: (i, 0))],
        out_specs=pl.BlockSpec((10000, D), lambda i: (i, 0)),
        out_shape=jax.ShapeDtypeStruct((NN, D), jnp.float32),
    )(x, v)


def _sum4_body(a_ref, b_ref, c_ref, d_ref, v_ref, o_ref):
    o_ref[...] = (a_ref[...] + b_ref[...] + c_ref[...] + d_ref[...]) * v_ref[...]


def _sum4scale(a, b, c, d, v):
    return pl.pallas_call(
        _sum4_body,
        grid=(5,),
        in_specs=[pl.BlockSpec((10000, D), lambda i: (i, 0))] * 4
        + [pl.BlockSpec((10000, 1), lambda i: (i, 0))],
        out_specs=pl.BlockSpec((10000, D), lambda i: (i, 0)),
        out_shape=jax.ShapeDtypeStruct((NN, D), jnp.float32),
    )(a, b, c, d, v)


def _loss_body(g_ref, o_ref):
    g = g_ref[...]
    u1, p1, n1 = g[0:B], g[B:2 * B], g[2 * B:3 * B]
    u2, p2, n2 = g[3 * B:4 * B], g[4 * B:5 * B], g[5 * B:6 * B]
    x = jnp.sum(u1 * (p1 - n1) + u2 * (p2 - n2), axis=1)
    ls = jnp.minimum(x, 0.0) - jnp.log1p(jnp.exp(-jnp.abs(x)))
    o_ref[...] = jnp.reshape(-jnp.mean(ls), (1, 1))


def _loss(g):
    out = pl.pallas_call(
        _loss_body,
        out_shape=jax.ShapeDtypeStruct((1, 1), jnp.float32),
    )(g)
    return out[0, 0]


def kernel(user_ids, pos_item_ids, neg_item_ids, e_u1, e_u2, e_i1, e_i2,
           adj_row, adj_col, adj_vals):
    i32 = jnp.int32
    srcp = adj_col.astype(i32)
    dstp = jnp.concatenate([adj_row[:HALF].astype(i32),
                            adj_row[HALF:].astype(i32) - NU])

    deg = jnp.maximum(_deg_call(dstp), 1.0)
    s_all = lax.rsqrt(deg)
    dinv = 1.0 / deg
    q_all = jnp.sqrt(deg) * 0.25

    tabs = [_scale0_call(e_u1, e_i1, e_u2, e_i2, s_all)]
    for _ in range(3):
        tabs.append(_layer_call(tabs[-1][0], tabs[-1][1], srcp, dstp, dinv))

    gathered = _fgather_call(tabs[0], tabs[1], tabs[2], tabs[3], q_all,
                             user_ids.astype(i32),
                             NU + pos_item_ids.astype(i32),
                             NU + neg_item_ids.astype(i32))
    return _loss(gathered)
